# baseline (speedup 1.0000x reference)
import functools
import math

import jax
import jax.numpy as jnp
from jax import lax
from jax.experimental import pallas as pl
from jax.experimental.pallas import tpu as pltpu
from jax.experimental.pallas import tpu_sc as plsc

NV = 10000
NE = 5000
NNZ = 160000
DIN = 256
DH = 512
NCLS = 40
NLAYERS = 4

NTILE = 16
NCHUNK = 4
CW = 128
NP_PAD = 10112
NE_PAD = 5120
PAIR_PAD = 163840
GROUPS = 80
GW = 128
EROWS = NE_PAD // NTILE
VROWS = NP_PAD // NTILE
GBLK = 8
SBLK = 64
RB = 128
NRB = NP_PAD // RB

_mesh = plsc.VectorSubcoreMesh(core_axis_name="c", subcore_axis_name="s",
                               num_cores=2, num_subcores=NTILE)


@functools.partial(
    pl.kernel,
    out_type=jax.ShapeDtypeStruct((NE_PAD, 16), jnp.float32),
    mesh=_mesh,
    compiler_params=pltpu.CompilerParams(needs_layout_passes=False),
    scratch_types=[
        pltpu.VMEM((GBLK, GW), jnp.int32),
        pltpu.VMEM((GW, CW), jnp.float32),
        pltpu.VMEM((SBLK, CW), jnp.float32),
        pltpu.VMEM((EROWS,), jnp.float32),
        pltpu.VMEM((SBLK, 16), jnp.float32),
        pltpu.VMEM_SHARED((NE_PAD, CW), jnp.float32),
    ],
)
def _sc_prep(eI, degE, s_out, eIv, ones_v, cnt_v, de_v, s_v, cntS):
    cid = lax.axis_index("c")
    sid = lax.axis_index("s")

    @pl.when(cid == 0)
    def _():
        r0 = sid * EROWS

        def init_ones(i, c):
            for k in range(CW // 16):
                ones_v[i, pl.ds(16 * k, 16)] = jnp.full((16,), 1.0,
                                                        jnp.float32)
            return c
        lax.fori_loop(0, GW, init_ones, 0)

        def init_zero(i, c):
            for k in range(CW // 16):
                cnt_v[i, pl.ds(16 * k, 16)] = jnp.zeros((16,), jnp.float32)
            return c
        lax.fori_loop(0, SBLK, init_zero, 0)
        for b in range(EROWS // SBLK):
            pltpu.sync_copy(cnt_v, cntS.at[pl.ds(r0 + b * SBLK, SBLK)])
        plsc.subcore_barrier()

        def add_group(g, c):
            pltpu.sync_copy(eI.at[sid, pl.ds(g * GBLK, GBLK)], eIv)

            def inner(j, c2):
                pltpu.sync_copy(ones_v, cntS.at[eIv.at[j]], add=True)
                return c2
            return lax.fori_loop(0, GBLK, inner, c)
        lax.fori_loop(0, GROUPS // GBLK, add_group, 0)
        plsc.subcore_barrier()

        pltpu.sync_copy(degE.at[pl.ds(r0, EROWS)], de_v)
        for b in range(EROWS // SBLK):
            pltpu.sync_copy(cntS.at[pl.ds(r0 + b * SBLK, SBLK)], cnt_v)

            def srow(r, carry, b=b):
                c16 = jnp.maximum(cnt_v[r, pl.ds(0, 16)], 1.0)
                d16 = plsc.load_gather(
                    de_v, [jnp.full((16,), b * SBLK + r, jnp.int32)])
                s_v[r, pl.ds(0, 16)] = d16 / c16
                return carry
            lax.fori_loop(0, SBLK, srow, 0)
            pltpu.sync_copy(s_v, s_out.at[pl.ds(r0 + b * SBLK, SBLK)])


_xe_t = [jax.ShapeDtypeStruct((NE_PAD, CW), jnp.float32) for _ in range(NCHUNK)]
_z_t = [jax.ShapeDtypeStruct((NP_PAD, CW), jnp.float32) for _ in range(NCHUNK)]


@functools.partial(
    pl.kernel,
    out_type=_xe_t,
    mesh=_mesh,
    compiler_params=pltpu.CompilerParams(needs_layout_passes=False),
    scratch_types=[
        pltpu.VMEM((GBLK, GW), jnp.int32),
        pltpu.VMEM((GBLK, GW), jnp.int32),
        pltpu.VMEM((GW, CW), jnp.float32),
        pltpu.VMEM((GW, CW), jnp.float32),
        pltpu.VMEM((SBLK, CW), jnp.float32),
        pltpu.VMEM((SBLK, 16), jnp.float32),
        pltpu.VMEM_SHARED((NE_PAD, CW), jnp.float32),
        pltpu.SemaphoreType.DMA,
        pltpu.SemaphoreType.DMA,
        pltpu.SemaphoreType.DMA,
        pltpu.SemaphoreType.DMA,
    ],
)
def _sc_phase_a(x0, x1, x2, x3, s, vI, eI, xo0, xo1, xo2, xo3,
                vIv, eIv, rows_v, rows_w, buf_v, s_v, Y,
                sem0, sem1, sem2, sem3):
    ssems = [sem2, sem3]
    xe_out = [xo0, xo1, xo2, xo3]
    cid = lax.axis_index("c")
    sid = lax.axis_index("s")
    r0e = sid * EROWS
    xcs = [x0, x1, x2, x3]
    bufs = [rows_v, rows_w]
    sems = [sem0, sem1]
    for c in range(NCHUNK):
        xc = xcs[c]
        xe_o = xe_out[c]

        @pl.when(cid == c // 2)
        def _(xc=xc, xe_o=xe_o):
            def zrow(i, carry):
                for k in range(CW // 16):
                    buf_v[i, pl.ds(16 * k, 16)] = jnp.zeros((16,), jnp.float32)
                return carry
            lax.fori_loop(0, SBLK, zrow, 0)
            for b in range(EROWS // SBLK):
                pltpu.sync_copy(buf_v, Y.at[pl.ds(r0e + b * SBLK, SBLK)])
            plsc.subcore_barrier()

            def body_a(g, carry):
                pltpu.sync_copy(vI.at[sid, pl.ds(g * GBLK, GBLK)], vIv)
                pltpu.sync_copy(eI.at[sid, pl.ds(g * GBLK, GBLK)], eIv)
                dg = [None] * GBLK
                ds_ = [None] * GBLK
                dg[0] = pltpu.async_copy(xc.at[vIv.at[0]], bufs[0], sems[0])
                for j in range(GBLK):
                    dg[j].wait()
                    if j + 1 < GBLK:
                        if j >= 1:
                            ds_[j - 1].wait()
                        dg[j + 1] = pltpu.async_copy(
                            xc.at[vIv.at[j + 1]], bufs[(j + 1) % 2],
                            sems[(j + 1) % 2])
                    ds_[j] = pltpu.async_copy(
                        bufs[j % 2], Y.at[eIv.at[j]], ssems[j % 2], add=True)
                ds_[GBLK - 2].wait()
                ds_[GBLK - 1].wait()
                return carry
            lax.fori_loop(0, GROUPS // GBLK, body_a, 0)
            plsc.subcore_barrier()

            for b in range(EROWS // SBLK):
                pltpu.sync_copy(s.at[pl.ds(r0e + b * SBLK, SBLK)], s_v)
                pltpu.sync_copy(Y.at[pl.ds(r0e + b * SBLK, SBLK)], buf_v)

                def body_s(r, carry):
                    sc = s_v[r, pl.ds(0, 16)]
                    for k in range(CW // 16):
                        buf_v[r, pl.ds(16 * k, 16)] = (
                            buf_v[r, pl.ds(16 * k, 16)] * sc)
                    return carry
                lax.fori_loop(0, SBLK, body_s, 0)
                pltpu.sync_copy(buf_v, xe_o.at[pl.ds(r0e + b * SBLK, SBLK)])
            plsc.subcore_barrier()


@functools.partial(
    pl.kernel,
    out_type=_z_t,
    mesh=_mesh,
    compiler_params=pltpu.CompilerParams(needs_layout_passes=False),
    scratch_types=[
        pltpu.VMEM((GBLK, GW), jnp.int32),
        pltpu.VMEM((GBLK, GW), jnp.int32),
        pltpu.VMEM((GW, CW), jnp.float32),
        pltpu.VMEM((GW, CW), jnp.float32),
        pltpu.VMEM_SHARED((NP_PAD, CW), jnp.float32),
        pltpu.SemaphoreType.DMA,
        pltpu.SemaphoreType.DMA,
        pltpu.SemaphoreType.DMA,
        pltpu.SemaphoreType.DMA,
    ],
)
def _sc_phase_b(xe0, xe1, xe2, xe3, vI, eI, zo0, zo1, zo2, zo3,
                vIv, eIv, rows_v, rows_w, Z, sem0, sem1, sem2, sem3):
    bufs = [rows_v, rows_w]
    sems = [sem0, sem1]
    ssems = [sem2, sem3]
    z_out = [zo0, zo1, zo2, zo3]
    cid = lax.axis_index("c")
    sid = lax.axis_index("s")
    r0n = sid * VROWS
    xes = [xe0, xe1, xe2, xe3]
    for c in range(NCHUNK):
        xe_i = xes[c]
        z_o = z_out[c]

        @pl.when(cid == c // 2)
        def _(xe_i=xe_i, z_o=z_o):
            def zrow(i, carry):
                for k in range(CW // 16):
                    rows_v[i, pl.ds(16 * k, 16)] = jnp.zeros((16,),
                                                             jnp.float32)
                return carry
            lax.fori_loop(0, GW, zrow, 0)
            for b in range(VROWS // GW):
                pltpu.sync_copy(rows_v, Z.at[pl.ds(r0n + b * GW, GW)])
            pltpu.sync_copy(rows_v.at[pl.ds(0, VROWS % GW)],
                            Z.at[pl.ds(r0n + (VROWS // GW) * GW, VROWS % GW)])
            plsc.subcore_barrier()

            def body_b(g, carry):
                pltpu.sync_copy(vI.at[sid, pl.ds(g * GBLK, GBLK)], vIv)
                pltpu.sync_copy(eI.at[sid, pl.ds(g * GBLK, GBLK)], eIv)
                dg = [None] * GBLK
                ds_ = [None] * GBLK
                dg[0] = pltpu.async_copy(xe_i.at[eIv.at[0]], bufs[0], sems[0])
                for j in range(GBLK):
                    dg[j].wait()
                    if j + 1 < GBLK:
                        if j >= 1:
                            ds_[j - 1].wait()
                        dg[j + 1] = pltpu.async_copy(
                            xe_i.at[eIv.at[j + 1]], bufs[(j + 1) % 2],
                            sems[(j + 1) % 2])
                    ds_[j] = pltpu.async_copy(
                        bufs[j % 2], Z.at[vIv.at[j]], ssems[j % 2], add=True)
                ds_[GBLK - 2].wait()
                ds_[GBLK - 1].wait()
                return carry
            lax.fori_loop(0, GROUPS // GBLK, body_b, 0)
            plsc.subcore_barrier()
            pltpu.sync_copy(Z.at[pl.ds(r0n, VROWS)], z_o.at[pl.ds(r0n, VROWS)])
            plsc.subcore_barrier()


def _in_proj_body(x_ref, w_ref, b_ref, o0, o1, o2, o3):
    y = lax.dot_general(x_ref[...], w_ref[...], (((1,), (1,)), ((), ())),
                        preferred_element_type=jnp.float32)
    y = jnp.maximum(y + b_ref[...], 0.0)
    for c, o in enumerate((o0, o1, o2, o3)):
        o[...] = y[:, c * CW:(c + 1) * CW]


_in_proj = pl.pallas_call(
    _in_proj_body,
    grid=(NRB,),
    in_specs=[
        pl.BlockSpec((RB, DIN), lambda i: (i, 0)),
        pl.BlockSpec((DH, DIN), lambda i: (0, 0)),
        pl.BlockSpec((1, DH), lambda i: (0, 0)),
    ],
    out_specs=[pl.BlockSpec((RB, CW), lambda i: (i, 0)) for _ in range(NCHUNK)],
    out_shape=[jax.ShapeDtypeStruct((NP_PAD, CW), jnp.float32)
               for _ in range(NCHUNK)],
)


def _combine_body(beta, z0, z1, z2, z3, x00, x01, x02, x03, dv_ref, w_ref,
                  o0, o1, o2, o3):
    zcat = jnp.concatenate([z0[...], z1[...], z2[...], z3[...]], axis=1)
    x0cat = jnp.concatenate([x00[...], x01[...], x02[...], x03[...]], axis=1)
    xi = 0.9 * (zcat * dv_ref[...]) + 0.1 * x0cat
    h = lax.dot_general(xi, w_ref[...], (((1,), (1,)), ((), ())),
                        preferred_element_type=jnp.float32)
    y = jnp.maximum((1.0 - beta) * xi + beta * h, 0.0)
    for c, o in enumerate((o0, o1, o2, o3)):
        o[...] = y[:, c * CW:(c + 1) * CW]


def _make_combine(beta):
    return pl.pallas_call(
        functools.partial(_combine_body, beta),
        grid=(NRB,),
        in_specs=(
            [pl.BlockSpec((RB, CW), lambda i: (i, 0)) for _ in range(NCHUNK)]
            + [pl.BlockSpec((RB, CW), lambda i: (i, 0)) for _ in range(NCHUNK)]
            + [pl.BlockSpec((RB, 1), lambda i: (i, 0)),
               pl.BlockSpec((DH, DH), lambda i: (0, 0))]
        ),
        out_specs=[pl.BlockSpec((RB, CW), lambda i: (i, 0))
                   for _ in range(NCHUNK)],
        out_shape=[jax.ShapeDtypeStruct((NP_PAD, CW), jnp.float32)
                   for _ in range(NCHUNK)],
    )


def _out_proj_body(x0, x1, x2, x3, w_ref, b_ref, o_ref):
    xcat = jnp.concatenate([x0[...], x1[...], x2[...], x3[...]], axis=1)
    y = lax.dot_general(xcat, w_ref[...], (((1,), (1,)), ((), ())),
                        preferred_element_type=jnp.float32)
    o_ref[...] = y + b_ref[...]


_out_proj = pl.pallas_call(
    _out_proj_body,
    grid=(NRB,),
    in_specs=(
        [pl.BlockSpec((RB, CW), lambda i: (i, 0)) for _ in range(NCHUNK)]
        + [pl.BlockSpec((NCLS, DH), lambda i: (0, 0)),
           pl.BlockSpec((1, NCLS), lambda i: (0, 0))]
    ),
    out_specs=pl.BlockSpec((RB, NCLS), lambda i: (i, 0)),
    out_shape=jax.ShapeDtypeStruct((NP_PAD, NCLS), jnp.float32),
)


def kernel(x, vertex, edges, degE, degV, W_in, b_in, W_convs, W_out, b_out):
    lamda, alpha = 0.5, 0.1
    del alpha

    xp = jnp.pad(x, ((0, NP_PAD - NV), (0, 0)))
    vP = jnp.concatenate(
        [vertex.astype(jnp.int32),
         jnp.full((PAIR_PAD - NNZ,), NV, jnp.int32)]).reshape(NTILE, GROUPS, GW)
    eP = jnp.concatenate(
        [edges.astype(jnp.int32),
         jnp.full((PAIR_PAD - NNZ,), NE, jnp.int32)]).reshape(NTILE, GROUPS, GW)
    degE_p = jnp.pad(degE[:, 0], (0, NE_PAD - NE))
    degV_p = jnp.pad(degV, ((0, NP_PAD - NV), (0, 0)))

    xc = _in_proj(xp, W_in, b_in.reshape(1, DH))
    s = _sc_prep(eP, degE_p)
    x0c = xc
    for i in range(NLAYERS):
        beta = math.log(lamda / (i + 1) + 1)
        xe = _sc_phase_a(*xc, s, vP, eP)
        zc = _sc_phase_b(*xe, vP, eP)
        xc = _make_combine(beta)(*zc, *x0c, degV_p, W_convs[i])
    out = _out_proj(*xc, W_out, b_out.reshape(1, NCLS))
    return out[:NV]


# speedup vs baseline: 1.0185x; 1.0185x over previous
import functools
import math

import jax
import jax.numpy as jnp
from jax import lax
from jax.experimental import pallas as pl
from jax.experimental.pallas import tpu as pltpu
from jax.experimental.pallas import tpu_sc as plsc

NV = 10000
NE = 5000
NNZ = 160000
DIN = 256
DH = 512
NCLS = 40
NLAYERS = 4

NTILE = 16
NCHUNK = 4
CW = 128
NP_PAD = 10112
NE_PAD = 5120
PAIR_PAD = 163840
GROUPS = 80
GW = 128
EROWS = NE_PAD // NTILE
VROWS = NP_PAD // NTILE
GBLK = 16
SBLK = 64
RB = 128
NRB = NP_PAD // RB

_mesh = plsc.VectorSubcoreMesh(core_axis_name="c", subcore_axis_name="s",
                               num_cores=2, num_subcores=NTILE)


@functools.partial(
    pl.kernel,
    out_type=jax.ShapeDtypeStruct((NE_PAD, 16), jnp.float32),
    mesh=_mesh,
    compiler_params=pltpu.CompilerParams(needs_layout_passes=False),
    scratch_types=[
        pltpu.VMEM((GBLK, GW), jnp.int32),
        pltpu.VMEM((GW, CW), jnp.float32),
        pltpu.VMEM((SBLK, CW), jnp.float32),
        pltpu.VMEM((EROWS,), jnp.float32),
        pltpu.VMEM((SBLK, 16), jnp.float32),
        pltpu.VMEM_SHARED((NE_PAD, CW), jnp.float32),
    ],
)
def _sc_prep(eI, degE, s_out, eIv, ones_v, cnt_v, de_v, s_v, cntS):
    cid = lax.axis_index("c")
    sid = lax.axis_index("s")

    @pl.when(cid == 0)
    def _():
        r0 = sid * EROWS

        def init_ones(i, c):
            for k in range(CW // 16):
                ones_v[i, pl.ds(16 * k, 16)] = jnp.full((16,), 1.0,
                                                        jnp.float32)
            return c
        lax.fori_loop(0, GW, init_ones, 0)

        def init_zero(i, c):
            for k in range(CW // 16):
                cnt_v[i, pl.ds(16 * k, 16)] = jnp.zeros((16,), jnp.float32)
            return c
        lax.fori_loop(0, SBLK, init_zero, 0)
        for b in range(EROWS // SBLK):
            pltpu.sync_copy(cnt_v, cntS.at[pl.ds(r0 + b * SBLK, SBLK)])
        plsc.subcore_barrier()

        def add_group(g, c):
            pltpu.sync_copy(eI.at[sid, pl.ds(g * GBLK, GBLK)], eIv)

            def inner(j, c2):
                pltpu.sync_copy(ones_v, cntS.at[eIv.at[j]], add=True)
                return c2
            return lax.fori_loop(0, GBLK, inner, c)
        lax.fori_loop(0, GROUPS // GBLK, add_group, 0)
        plsc.subcore_barrier()

        pltpu.sync_copy(degE.at[pl.ds(r0, EROWS)], de_v)
        for b in range(EROWS // SBLK):
            pltpu.sync_copy(cntS.at[pl.ds(r0 + b * SBLK, SBLK)], cnt_v)

            def srow(r, carry, b=b):
                c16 = jnp.maximum(cnt_v[r, pl.ds(0, 16)], 1.0)
                d16 = plsc.load_gather(
                    de_v, [jnp.full((16,), b * SBLK + r, jnp.int32)])
                s_v[r, pl.ds(0, 16)] = d16 / c16
                return carry
            lax.fori_loop(0, SBLK, srow, 0)
            pltpu.sync_copy(s_v, s_out.at[pl.ds(r0 + b * SBLK, SBLK)])


_xe_t = [jax.ShapeDtypeStruct((NE_PAD, CW), jnp.float32) for _ in range(NCHUNK)]
_z_t = [jax.ShapeDtypeStruct((NP_PAD, CW), jnp.float32) for _ in range(NCHUNK)]


@functools.partial(
    pl.kernel,
    out_type=_xe_t,
    mesh=_mesh,
    compiler_params=pltpu.CompilerParams(needs_layout_passes=False),
    scratch_types=[
        pltpu.VMEM((GBLK, GW), jnp.int32),
        pltpu.VMEM((GBLK, GW), jnp.int32),
        pltpu.VMEM((GW, CW), jnp.float32),
        pltpu.VMEM((GW, CW), jnp.float32),
        pltpu.VMEM((SBLK, CW), jnp.float32),
        pltpu.VMEM((SBLK, 16), jnp.float32),
        pltpu.VMEM_SHARED((NE_PAD, CW), jnp.float32),
        pltpu.SemaphoreType.DMA,
        pltpu.SemaphoreType.DMA,
        pltpu.SemaphoreType.DMA,
        pltpu.SemaphoreType.DMA,
    ],
)
def _sc_phase_a(x0, x1, x2, x3, s, vI, eI, xo0, xo1, xo2, xo3,
                vIv, eIv, rows_v, rows_w, buf_v, s_v, Y,
                sem0, sem1, sem2, sem3):
    ssems = [sem2, sem3]
    xe_out = [xo0, xo1, xo2, xo3]
    cid = lax.axis_index("c")
    sid = lax.axis_index("s")
    r0e = sid * EROWS
    xcs = [x0, x1, x2, x3]
    bufs = [rows_v, rows_w]
    sems = [sem0, sem1]
    for c in range(NCHUNK):
        xc = xcs[c]
        xe_o = xe_out[c]

        @pl.when(cid == c // 2)
        def _(xc=xc, xe_o=xe_o):
            def zrow(i, carry):
                for k in range(CW // 16):
                    buf_v[i, pl.ds(16 * k, 16)] = jnp.zeros((16,), jnp.float32)
                return carry
            lax.fori_loop(0, SBLK, zrow, 0)
            for b in range(EROWS // SBLK):
                pltpu.sync_copy(buf_v, Y.at[pl.ds(r0e + b * SBLK, SBLK)])
            plsc.subcore_barrier()

            def body_a(g, carry):
                pltpu.sync_copy(vI.at[sid, pl.ds(g * GBLK, GBLK)], vIv)
                pltpu.sync_copy(eI.at[sid, pl.ds(g * GBLK, GBLK)], eIv)
                dg = [None] * GBLK
                ds_ = [None] * GBLK
                dg[0] = pltpu.async_copy(xc.at[vIv.at[0]], bufs[0], sems[0])
                for j in range(GBLK):
                    dg[j].wait()
                    if j + 1 < GBLK:
                        if j >= 1:
                            ds_[j - 1].wait()
                        dg[j + 1] = pltpu.async_copy(
                            xc.at[vIv.at[j + 1]], bufs[(j + 1) % 2],
                            sems[(j + 1) % 2])
                    ds_[j] = pltpu.async_copy(
                        bufs[j % 2], Y.at[eIv.at[j]], ssems[j % 2], add=True)
                ds_[GBLK - 2].wait()
                ds_[GBLK - 1].wait()
                return carry
            lax.fori_loop(0, GROUPS // GBLK, body_a, 0)
            plsc.subcore_barrier()

            for b in range(EROWS // SBLK):
                pltpu.sync_copy(s.at[pl.ds(r0e + b * SBLK, SBLK)], s_v)
                pltpu.sync_copy(Y.at[pl.ds(r0e + b * SBLK, SBLK)], buf_v)

                def body_s(r, carry):
                    sc = s_v[r, pl.ds(0, 16)]
                    for k in range(CW // 16):
                        buf_v[r, pl.ds(16 * k, 16)] = (
                            buf_v[r, pl.ds(16 * k, 16)] * sc)
                    return carry
                lax.fori_loop(0, SBLK, body_s, 0)
                pltpu.sync_copy(buf_v, xe_o.at[pl.ds(r0e + b * SBLK, SBLK)])
            plsc.subcore_barrier()


@functools.partial(
    pl.kernel,
    out_type=_z_t,
    mesh=_mesh,
    compiler_params=pltpu.CompilerParams(needs_layout_passes=False),
    scratch_types=[
        pltpu.VMEM((GBLK, GW), jnp.int32),
        pltpu.VMEM((GBLK, GW), jnp.int32),
        pltpu.VMEM((GW, CW), jnp.float32),
        pltpu.VMEM((GW, CW), jnp.float32),
        pltpu.VMEM_SHARED((NP_PAD, CW), jnp.float32),
        pltpu.SemaphoreType.DMA,
        pltpu.SemaphoreType.DMA,
        pltpu.SemaphoreType.DMA,
        pltpu.SemaphoreType.DMA,
    ],
)
def _sc_phase_b(xe0, xe1, xe2, xe3, vI, eI, zo0, zo1, zo2, zo3,
                vIv, eIv, rows_v, rows_w, Z, sem0, sem1, sem2, sem3):
    bufs = [rows_v, rows_w]
    sems = [sem0, sem1]
    ssems = [sem2, sem3]
    z_out = [zo0, zo1, zo2, zo3]
    cid = lax.axis_index("c")
    sid = lax.axis_index("s")
    r0n = sid * VROWS
    xes = [xe0, xe1, xe2, xe3]
    for c in range(NCHUNK):
        xe_i = xes[c]
        z_o = z_out[c]

        @pl.when(cid == c // 2)
        def _(xe_i=xe_i, z_o=z_o):
            def zrow(i, carry):
                for k in range(CW // 16):
                    rows_v[i, pl.ds(16 * k, 16)] = jnp.zeros((16,),
                                                             jnp.float32)
                return carry
            lax.fori_loop(0, GW, zrow, 0)
            for b in range(VROWS // GW):
                pltpu.sync_copy(rows_v, Z.at[pl.ds(r0n + b * GW, GW)])
            pltpu.sync_copy(rows_v.at[pl.ds(0, VROWS % GW)],
                            Z.at[pl.ds(r0n + (VROWS // GW) * GW, VROWS % GW)])
            plsc.subcore_barrier()

            def body_b(g, carry):
                pltpu.sync_copy(vI.at[sid, pl.ds(g * GBLK, GBLK)], vIv)
                pltpu.sync_copy(eI.at[sid, pl.ds(g * GBLK, GBLK)], eIv)
                dg = [None] * GBLK
                ds_ = [None] * GBLK
                dg[0] = pltpu.async_copy(xe_i.at[eIv.at[0]], bufs[0], sems[0])
                for j in range(GBLK):
                    dg[j].wait()
                    if j + 1 < GBLK:
                        if j >= 1:
                            ds_[j - 1].wait()
                        dg[j + 1] = pltpu.async_copy(
                            xe_i.at[eIv.at[j + 1]], bufs[(j + 1) % 2],
                            sems[(j + 1) % 2])
                    ds_[j] = pltpu.async_copy(
                        bufs[j % 2], Z.at[vIv.at[j]], ssems[j % 2], add=True)
                ds_[GBLK - 2].wait()
                ds_[GBLK - 1].wait()
                return carry
            lax.fori_loop(0, GROUPS // GBLK, body_b, 0)
            plsc.subcore_barrier()
            pltpu.sync_copy(Z.at[pl.ds(r0n, VROWS)], z_o.at[pl.ds(r0n, VROWS)])
            plsc.subcore_barrier()


def _in_proj_body(x_ref, w_ref, b_ref, o0, o1, o2, o3):
    y = lax.dot_general(x_ref[...], w_ref[...], (((1,), (1,)), ((), ())),
                        preferred_element_type=jnp.float32)
    y = jnp.maximum(y + b_ref[...], 0.0)
    for c, o in enumerate((o0, o1, o2, o3)):
        o[...] = y[:, c * CW:(c + 1) * CW]


_in_proj = pl.pallas_call(
    _in_proj_body,
    grid=(NRB,),
    in_specs=[
        pl.BlockSpec((RB, DIN), lambda i: (i, 0)),
        pl.BlockSpec((DH, DIN), lambda i: (0, 0)),
        pl.BlockSpec((1, DH), lambda i: (0, 0)),
    ],
    out_specs=[pl.BlockSpec((RB, CW), lambda i: (i, 0)) for _ in range(NCHUNK)],
    out_shape=[jax.ShapeDtypeStruct((NP_PAD, CW), jnp.float32)
               for _ in range(NCHUNK)],
)


def _combine_body(beta, z0, z1, z2, z3, x00, x01, x02, x03, dv_ref, w_ref,
                  o0, o1, o2, o3):
    zcat = jnp.concatenate([z0[...], z1[...], z2[...], z3[...]], axis=1)
    x0cat = jnp.concatenate([x00[...], x01[...], x02[...], x03[...]], axis=1)
    xi = 0.9 * (zcat * dv_ref[...]) + 0.1 * x0cat
    h = lax.dot_general(xi, w_ref[...], (((1,), (1,)), ((), ())),
                        preferred_element_type=jnp.float32)
    y = jnp.maximum((1.0 - beta) * xi + beta * h, 0.0)
    for c, o in enumerate((o0, o1, o2, o3)):
        o[...] = y[:, c * CW:(c + 1) * CW]


def _make_combine(beta):
    return pl.pallas_call(
        functools.partial(_combine_body, beta),
        grid=(NRB,),
        in_specs=(
            [pl.BlockSpec((RB, CW), lambda i: (i, 0)) for _ in range(NCHUNK)]
            + [pl.BlockSpec((RB, CW), lambda i: (i, 0)) for _ in range(NCHUNK)]
            + [pl.BlockSpec((RB, 1), lambda i: (i, 0)),
               pl.BlockSpec((DH, DH), lambda i: (0, 0))]
        ),
        out_specs=[pl.BlockSpec((RB, CW), lambda i: (i, 0))
                   for _ in range(NCHUNK)],
        out_shape=[jax.ShapeDtypeStruct((NP_PAD, CW), jnp.float32)
                   for _ in range(NCHUNK)],
    )


def _out_proj_body(x0, x1, x2, x3, w_ref, b_ref, o_ref):
    xcat = jnp.concatenate([x0[...], x1[...], x2[...], x3[...]], axis=1)
    y = lax.dot_general(xcat, w_ref[...], (((1,), (1,)), ((), ())),
                        preferred_element_type=jnp.float32)
    o_ref[...] = y + b_ref[...]


_out_proj = pl.pallas_call(
    _out_proj_body,
    grid=(NRB,),
    in_specs=(
        [pl.BlockSpec((RB, CW), lambda i: (i, 0)) for _ in range(NCHUNK)]
        + [pl.BlockSpec((NCLS, DH), lambda i: (0, 0)),
           pl.BlockSpec((1, NCLS), lambda i: (0, 0))]
    ),
    out_specs=pl.BlockSpec((RB, NCLS), lambda i: (i, 0)),
    out_shape=jax.ShapeDtypeStruct((NP_PAD, NCLS), jnp.float32),
)


def kernel(x, vertex, edges, degE, degV, W_in, b_in, W_convs, W_out, b_out):
    lamda, alpha = 0.5, 0.1
    del alpha

    xp = jnp.pad(x, ((0, NP_PAD - NV), (0, 0)))
    vP = jnp.concatenate(
        [vertex.astype(jnp.int32),
         jnp.full((PAIR_PAD - NNZ,), NV, jnp.int32)]).reshape(NTILE, GROUPS, GW)
    eP = jnp.concatenate(
        [edges.astype(jnp.int32),
         jnp.full((PAIR_PAD - NNZ,), NE, jnp.int32)]).reshape(NTILE, GROUPS, GW)
    degE_p = jnp.pad(degE[:, 0], (0, NE_PAD - NE))
    degV_p = jnp.pad(degV, ((0, NP_PAD - NV), (0, 0)))

    xc = _in_proj(xp, W_in, b_in.reshape(1, DH))
    s = _sc_prep(eP, degE_p)
    x0c = xc
    for i in range(NLAYERS):
        beta = math.log(lamda / (i + 1) + 1)
        xe = _sc_phase_a(*xc, s, vP, eP)
        zc = _sc_phase_b(*xe, vP, eP)
        xc = _make_combine(beta)(*zc, *x0c, degV_p, W_convs[i])
    out = _out_proj(*xc, W_out, b_out.reshape(1, NCLS))
    return out[:NV]
